# baseline (speedup 1.0000x reference)
import jax
import jax.numpy as jnp
import numpy as np
from jax import lax
from jax.experimental import pallas as pl
from jax.experimental.pallas import tpu as pltpu
from jax.experimental.pallas import tpu_sc as plsc

NC, NS, L = 2, 16, 16
NW = NC * NS
HID = 128
ROWS = 65
NF = 4
B, S = 4096, 200
N = B * S
TPW = N // NW
CHUNK = 256
NCHUNK = TPW // CHUNK
TAB_WORDS = NF * ROWS * HID


def _body(ids_hbm, tab_hbm, out_hbm, tab_v, ids_v, out_v, sem_tab, sem_ids,
          sem_out):
    wid = lax.axis_index("s") * NC + lax.axis_index("c")
    base = wid * TPW

    pltpu.async_copy(tab_hbm, tab_v, sem_tab).wait()

    def load_ids(g, slot):
        return pltpu.async_copy(
            ids_hbm.at[pl.ds((base + g * CHUNK) * NF, CHUNK * NF)],
            ids_v.at[pl.ds(slot * (CHUNK * NF), CHUNK * NF)], sem_ids)

    def store_out(g, slot):
        return pltpu.async_copy(
            out_v.at[pl.ds(slot * (CHUNK * HID), CHUNK * HID)],
            out_hbm.at[pl.ds((base + g * CHUNK) * HID, CHUNK * HID)],
            sem_out)

    load_ids(0, 0).wait()

    iota = jnp.arange(L, dtype=jnp.int32)
    iota4 = iota * NF
    iotah = iota * HID

    def chunk_body(g, _):
        slot = lax.rem(g, 2)

        @pl.when(g + 1 < NCHUNK)
        def _():
            load_ids(g + 1, 1 - slot)

        def tok_body(q, _):
            offs = [
                (plsc.load_gather(
                    ids_v, [iota4 + (slot * (CHUNK * NF) + q * (L * NF) + f)])
                 * HID + f * (ROWS * HID))
                for f in range(NF)
            ]
            oidx = iotah + (slot * (CHUNK * HID) + q * (L * HID))
            for h in range(HID):
                acc = (plsc.load_gather(tab_v, [offs[0] + h]) +
                       plsc.load_gather(tab_v, [offs[1] + h]) +
                       plsc.load_gather(tab_v, [offs[2] + h]) +
                       plsc.load_gather(tab_v, [offs[3] + h]))
                plsc.store_scatter(out_v, [oidx + h], acc)
            return 0

        lax.fori_loop(0, CHUNK // L, tok_body, 0)

        @pl.when(g >= 2)
        def _():
            pltpu.make_async_copy(
                out_v.at[pl.ds(0, CHUNK * HID)],
                out_hbm.at[pl.ds(0, CHUNK * HID)], sem_out).wait()

        store_out(g, slot)

        @pl.when(g + 1 < NCHUNK)
        def _():
            pltpu.make_async_copy(
                ids_v.at[pl.ds(0, CHUNK * NF)],
                ids_hbm.at[pl.ds(0, CHUNK * NF)], sem_ids).wait()
        return 0

    lax.fori_loop(0, NCHUNK, chunk_body, 0)

    for _ in range(2):
        pltpu.make_async_copy(out_v.at[pl.ds(0, CHUNK * HID)],
                              out_hbm.at[pl.ds(0, CHUNK * HID)],
                              sem_out).wait()


@jax.jit
def _run(ids_flat, tab_flat):
    mesh = plsc.VectorSubcoreMesh(core_axis_name="c", subcore_axis_name="s",
                                  num_cores=NC, num_subcores=NS)
    return pl.kernel(
        _body,
        out_type=jax.ShapeDtypeStruct((N * HID,), jnp.float32),
        mesh=mesh,
        scratch_types=[
            pltpu.VMEM((TAB_WORDS,), jnp.float32),
            pltpu.VMEM((2 * CHUNK * NF,), jnp.int32),
            pltpu.VMEM((2 * CHUNK * HID,), jnp.float32),
            pltpu.SemaphoreType.DMA,
            pltpu.SemaphoreType.DMA,
            pltpu.SemaphoreType.DMA,
        ],
        compiler_params=pltpu.CompilerParams(needs_layout_passes=False),
    )(ids_flat, tab_flat)


def kernel(noise_ids, W0, W1, W2, W3):
    ids_flat = noise_ids.reshape(N * NF)
    tab_flat = jnp.concatenate([W0, W1, W2, W3], axis=0).reshape(-1)
    out = _run(ids_flat, tab_flat)
    return out.reshape(B, S, HID)


# speedup vs baseline: 6.1933x; 6.1933x over previous
import jax
import jax.numpy as jnp
from jax import lax
from jax.experimental import pallas as pl
from jax.experimental.pallas import tpu as pltpu
from jax.experimental.pallas import tpu_sc as plsc

NC, NS, L = 2, 16, 16
NW = NC * NS
HID = 128
ROWS = 65
NF = 4
B, S = 4096, 200
RPW = B // NW
TAB_WORDS = NF * ROWS * HID


def _body(ids_hbm, tab_hbm, out_hbm, tab_v, ids_v, out_v, sem_tab, sem_ids,
          sem_out):
    wid = lax.axis_index("s") * NC + lax.axis_index("c")
    row0 = wid * RPW

    pltpu.async_copy(tab_hbm, tab_v, sem_tab).wait()

    def load_ids(g, slot):
        return pltpu.async_copy(
            ids_hbm.at[pl.ds((row0 + g) * (S * NF), S * NF)],
            ids_v.at[pl.ds(slot * (S * NF), S * NF)], sem_ids)

    def store_out(g, slot):
        return pltpu.async_copy(
            out_v.at[slot], out_hbm.at[row0 + g], sem_out)

    load_ids(0, 0).wait()

    offpat = (jnp.arange(L, dtype=jnp.int32) % NF) * (ROWS * HID)

    def chunk_body(g, _):
        slot = lax.rem(g, 2)

        @pl.when(g + 1 < RPW)
        def _():
            load_ids(g + 1, 1 - slot)

        @plsc.parallel_loop(0, S // 4, unroll=2)
        def tok_body(q):
            offs = ids_v[pl.ds(slot * (S * NF) + q * L, L)] * HID + offpat
            for j in range(4):
                t = q * 4 + j
                for c in range(HID // L):
                    t0 = tab_v[pl.ds(offs[4 * j + 0] + c * L, L)]
                    t1 = tab_v[pl.ds(offs[4 * j + 1] + c * L, L)]
                    t2 = tab_v[pl.ds(offs[4 * j + 2] + c * L, L)]
                    t3 = tab_v[pl.ds(offs[4 * j + 3] + c * L, L)]
                    out_v[slot, t, pl.ds(c * L, L)] = (t0 + t1) + (t2 + t3)

        @pl.when(g >= 2)
        def _():
            pltpu.make_async_copy(out_v.at[0], out_hbm.at[0],
                                  sem_out).wait()

        store_out(g, slot)

        @pl.when(g + 1 < RPW)
        def _():
            pltpu.make_async_copy(
                ids_v.at[pl.ds(0, S * NF)],
                ids_hbm.at[pl.ds(0, S * NF)], sem_ids).wait()
        return 0

    lax.fori_loop(0, RPW, chunk_body, 0)

    for _ in range(2):
        pltpu.make_async_copy(out_v.at[0], out_hbm.at[0], sem_out).wait()


@jax.jit
def _run(ids_flat, tab_flat):
    mesh = plsc.VectorSubcoreMesh(core_axis_name="c", subcore_axis_name="s",
                                  num_cores=NC, num_subcores=NS)
    return pl.kernel(
        _body,
        out_type=jax.ShapeDtypeStruct((B, S, HID), jnp.float32),
        mesh=mesh,
        scratch_types=[
            pltpu.VMEM((TAB_WORDS,), jnp.float32),
            pltpu.VMEM((2 * S * NF,), jnp.int32),
            pltpu.VMEM((2, S, HID), jnp.float32),
            pltpu.SemaphoreType.DMA,
            pltpu.SemaphoreType.DMA,
            pltpu.SemaphoreType.DMA,
        ],
        compiler_params=pltpu.CompilerParams(needs_layout_passes=False),
    )(ids_flat, tab_flat)


def kernel(noise_ids, W0, W1, W2, W3):
    ids_flat = noise_ids.reshape(B * S * NF)
    tab_flat = jnp.concatenate([W0, W1, W2, W3], axis=0).reshape(-1)
    return _run(ids_flat, tab_flat)
